# baseline (speedup 1.0000x reference)
import functools

import jax
import jax.numpy as jnp
from jax import lax
from jax.experimental import pallas as pl
from jax.experimental.pallas import tpu as pltpu
from jax.experimental.pallas import tpu_sc as plsc

N = 10000
E = 320000
D_IN = 128
D_H = 64
D2 = 16
D_P = 128
NC = 2
NS = 16
NT = NC * NS
B = 128
CH = -(-((E + NT * B - 1) // (NT * B)) // 2) * 2
CH2 = CH // 2
E_PAD = NT * CH * B
N_PAD = 10112
RPT = N_PAD // NS

_f32 = jnp.float32


def _sc_scatter_add(D, gather):
    mesh = plsc.VectorSubcoreMesh(core_axis_name="c", subcore_axis_name="s")
    scratch = [
        pltpu.VMEM((CH2, B), jnp.int32),
        pltpu.VMEM((CH2, B), jnp.int32),
        pltpu.VMEM((B, D), _f32),
        pltpu.VMEM((B, D), _f32),
        pltpu.SemaphoreType.DMA,
        pltpu.SemaphoreType.DMA,
        pltpu.VMEM_SHARED((N_PAD, D), _f32),
    ]

    @functools.partial(
        pl.kernel,
        out_type=jax.ShapeDtypeStruct((NC, N_PAD, D), _f32),
        mesh=mesh,
        scratch_types=scratch,
    )
    def k(vals_hbm, srcr_hbm, dstr_hbm, zeros_hbm, out_hbm,
          dst_v, src_v, rows_a, rows_b, sem_a, sem_b, accum):
        c = lax.axis_index("c")
        s = lax.axis_index("s")
        t = c * NS + s
        if not gather:
            pltpu.sync_copy(vals_hbm, rows_a)
        pltpu.sync_copy(zeros_hbm.at[pl.ds(s * RPT, RPT)],
                        accum.at[pl.ds(s * RPT, RPT)])
        plsc.subcore_barrier()

        for h in range(2):
            pltpu.sync_copy(dstr_hbm.at[t].at[h], dst_v)
            if gather:
                pltpu.sync_copy(srcr_hbm.at[t].at[h], src_v)

                def gat(j, buf, sem):
                    return pltpu.async_copy(vals_hbm.at[src_v.at[j]],
                                            buf, sem)

                def wait(j, buf, sem):
                    pltpu.make_async_copy(vals_hbm.at[src_v.at[j]],
                                          buf, sem).wait()

                gat(0, rows_a, sem_a)

                def step(i, carry):
                    ja = 2 * i
                    gat(ja + 1, rows_b, sem_b)
                    wait(ja, rows_a, sem_a)
                    pltpu.sync_copy(rows_a, accum.at[dst_v.at[ja]], add=True)

                    @pl.when(ja + 2 < CH2)
                    def _():
                        gat(ja + 2, rows_a, sem_a)

                    wait(ja + 1, rows_b, sem_b)
                    pltpu.sync_copy(rows_b, accum.at[dst_v.at[ja + 1]],
                                    add=True)
                    return carry

                lax.fori_loop(0, CH2 // 2, step, 0)
            else:
                def step(j, carry):
                    pltpu.sync_copy(rows_a, accum.at[dst_v.at[j]], add=True)
                    return carry

                lax.fori_loop(0, CH2, step, 0)

        plsc.subcore_barrier()
        pltpu.sync_copy(accum.at[pl.ds(s * RPT, RPT)],
                        out_hbm.at[c].at[pl.ds(s * RPT, RPT)])

    return k


_sc_hist = _sc_scatter_add(D_P, gather=False)
_sc_scatter = _sc_scatter_add(D_P, gather=True)


def _tc_matmul(x_ref, w_ref, o_ref):
    o_ref[...] = jnp.dot(x_ref[...], w_ref[...],
                         preferred_element_type=_f32)


def _tc_scale(degp_ref, xw_ref, y1_ref, dinv_ref):
    deg = 1.0 + degp_ref[0][:, :D2] + degp_ref[1][:, :D2]
    dinv = lax.rsqrt(deg)
    dinv_ref[...] = dinv
    y1_ref[...] = xw_ref[...] * dinv[:N, 0:1]


def _tc_mid(segp_ref, y1_ref, dinv_ref, b1_ref, g1_ref, be1_ref, z_ref):
    dcol = dinv_ref[...][:N, 0:1]
    seg = segp_ref[0][:N, :D_H] + segp_ref[1][:N, :D_H] + y1_ref[...]
    hpre = seg * dcol + b1_ref[...]
    mean = jnp.mean(hpre, axis=0, keepdims=True)
    var = jnp.mean((hpre - mean) ** 2, axis=0, keepdims=True)
    h = (hpre - mean) * lax.rsqrt(var + 1e-5) * g1_ref[...] + be1_ref[...]
    h = jnp.maximum(h, 0.0)
    z_ref[...] = h * dcol


def _tc_final(segp_ref, z_ref, dinv_ref, w2_ref, b2_ref, o_ref):
    dcol = dinv_ref[...][:N, 0:1]
    seg = (segp_ref[0][:N, :D_H] + segp_ref[1][:N, :D_H] + z_ref[...]) * dcol
    o_ref[...] = jnp.dot(seg, w2_ref[...],
                         preferred_element_type=_f32) + b2_ref[...]


def kernel(x, edge_index, W1, b1, gamma1, beta1, W2, b2):
    src = edge_index[0]
    dst = edge_index[1]
    pad = E_PAD - E
    src_r = jnp.concatenate(
        [src, jnp.zeros((pad,), jnp.int32)]).reshape(NT, 2, CH2, B)
    dst_r = jnp.concatenate(
        [dst, jnp.full((pad,), N, jnp.int32)]).reshape(NT, 2, CH2, B)
    z128 = jnp.zeros((N_PAD, D_P), _f32)
    ones128 = jnp.ones((B, D_P), _f32)
    w2p = jnp.pad(W2, ((0, 0), (0, D2 - W2.shape[1])))
    b1r = b1.reshape(1, D_H)
    g1r = gamma1.reshape(1, D_H)
    be1r = beta1.reshape(1, D_H)
    b2r = jnp.pad(b2, (0, D2 - b2.shape[0])).reshape(1, D2)

    degp = _sc_hist(ones128, src_r, dst_r, z128)
    xw = pl.pallas_call(
        _tc_matmul,
        out_shape=jax.ShapeDtypeStruct((N, D_H), _f32),
    )(x, W1)

    y1, dinv = pl.pallas_call(
        _tc_scale,
        out_shape=[jax.ShapeDtypeStruct((N, D_H), _f32),
                   jax.ShapeDtypeStruct((N_PAD, D2), _f32)],
    )(degp, xw)

    y1p = jnp.pad(y1, ((0, 0), (0, D_P - D_H)))
    seg1p = _sc_scatter(y1p, src_r, dst_r, z128)

    z = pl.pallas_call(
        _tc_mid,
        out_shape=jax.ShapeDtypeStruct((N, D_H), _f32),
    )(seg1p, y1, dinv, b1r, g1r, be1r)

    zp = jnp.pad(z, ((0, 0), (0, D_P - D_H)))
    seg2p = _sc_scatter(zp, src_r, dst_r, z128)

    out8 = pl.pallas_call(
        _tc_final,
        out_shape=jax.ShapeDtypeStruct((N, D2), _f32),
    )(seg2p, z, dinv, w2p, b2r)

    return out8[:, :2]


# speedup vs baseline: 1.0007x; 1.0007x over previous
import functools

import jax
import jax.numpy as jnp
from jax import lax
from jax.experimental import pallas as pl
from jax.experimental.pallas import tpu as pltpu
from jax.experimental.pallas import tpu_sc as plsc

N = 10000
E = 320000
D_IN = 128
D_H = 64
D2 = 16
D_P = 128
NC = 2
NS = 16
NT = NC * NS
B = 128
CH = -(-((E + NT * B - 1) // (NT * B)) // 2) * 2
CH2 = CH // 2
E_PAD = NT * CH * B
N_PAD = 10112
RPT = N_PAD // NS

_f32 = jnp.float32


def _sc_scatter_add(D, gather):
    mesh = plsc.VectorSubcoreMesh(core_axis_name="c", subcore_axis_name="s")
    scratch = [
        pltpu.VMEM((CH2, B), jnp.int32),
        pltpu.VMEM((CH2, B), jnp.int32),
        pltpu.VMEM((B, D), _f32),
        pltpu.VMEM((B, D), _f32),
        pltpu.SemaphoreType.DMA,
        pltpu.SemaphoreType.DMA,
        pltpu.VMEM_SHARED((N_PAD, D), _f32),
    ]

    @functools.partial(
        pl.kernel,
        out_type=jax.ShapeDtypeStruct((NC, N_PAD, D), _f32),
        mesh=mesh,
        scratch_types=scratch,
    )
    def k(vals_hbm, srcr_hbm, dstr_hbm, zeros_hbm, out_hbm,
          dst_v, src_v, rows_a, rows_b, sem_a, sem_b, accum):
        c = lax.axis_index("c")
        s = lax.axis_index("s")
        t = c * NS + s
        if not gather:
            pltpu.sync_copy(vals_hbm, rows_a)
        pltpu.sync_copy(zeros_hbm.at[pl.ds(s * RPT, RPT)],
                        accum.at[pl.ds(s * RPT, RPT)])
        plsc.subcore_barrier()

        for h in range(2):
            pltpu.sync_copy(dstr_hbm.at[t].at[h], dst_v)
            if gather:
                pltpu.sync_copy(srcr_hbm.at[t].at[h], src_v)

                def gat(j, buf, sem):
                    return pltpu.async_copy(vals_hbm.at[src_v.at[j]],
                                            buf, sem)

                def wait(j, buf, sem):
                    pltpu.make_async_copy(vals_hbm.at[src_v.at[j]],
                                          buf, sem).wait()

                def scat(buf, j):
                    pltpu.sync_copy(buf, accum.at[dst_v.at[j]], add=True)

                gat(0, rows_a, sem_a)

                def step(i, carry):
                    ja = 2 * i
                    gat(ja + 1, rows_b, sem_b)
                    wait(ja, rows_a, sem_a)
                    scat(rows_a, ja)
                    gat(ja + 2, rows_a, sem_a)
                    wait(ja + 1, rows_b, sem_b)
                    scat(rows_b, ja + 1)
                    return carry

                lax.fori_loop(0, CH2 // 2 - 1, step, 0)
                ja = CH2 - 2
                gat(ja + 1, rows_b, sem_b)
                wait(ja, rows_a, sem_a)
                scat(rows_a, ja)
                wait(ja + 1, rows_b, sem_b)
                scat(rows_b, ja + 1)
            else:
                def step(j, carry):
                    pltpu.sync_copy(rows_a, accum.at[dst_v.at[j]], add=True)
                    return carry

                lax.fori_loop(0, CH2, step, 0)

        plsc.subcore_barrier()
        pltpu.sync_copy(accum.at[pl.ds(s * RPT, RPT)],
                        out_hbm.at[c].at[pl.ds(s * RPT, RPT)])

    return k


_sc_hist = _sc_scatter_add(D_P, gather=False)
_sc_scatter = _sc_scatter_add(D_P, gather=True)


def _tc_matmul(x_ref, w_ref, o_ref):
    o_ref[...] = jnp.dot(x_ref[...], w_ref[...],
                         preferred_element_type=_f32)


def _tc_scale(degp_ref, xw_ref, y1_ref, dinv_ref):
    deg = 1.0 + degp_ref[0][:, :D2] + degp_ref[1][:, :D2]
    dinv = lax.rsqrt(deg)
    dinv_ref[...] = dinv
    y1_ref[...] = xw_ref[...] * dinv[:N, 0:1]


def _tc_mid(segp_ref, y1_ref, dinv_ref, b1_ref, g1_ref, be1_ref, z_ref):
    dcol = dinv_ref[...][:N, 0:1]
    seg = segp_ref[0][:N, :D_H] + segp_ref[1][:N, :D_H] + y1_ref[...]
    hpre = seg * dcol + b1_ref[...]
    mean = jnp.mean(hpre, axis=0, keepdims=True)
    var = jnp.mean((hpre - mean) ** 2, axis=0, keepdims=True)
    h = (hpre - mean) * lax.rsqrt(var + 1e-5) * g1_ref[...] + be1_ref[...]
    h = jnp.maximum(h, 0.0)
    z_ref[...] = h * dcol


def _tc_final(segp_ref, z_ref, dinv_ref, w2_ref, b2_ref, o_ref):
    dcol = dinv_ref[...][:N, 0:1]
    seg = (segp_ref[0][:N, :D_H] + segp_ref[1][:N, :D_H] + z_ref[...]) * dcol
    o_ref[...] = jnp.dot(seg, w2_ref[...],
                         preferred_element_type=_f32) + b2_ref[...]


def kernel(x, edge_index, W1, b1, gamma1, beta1, W2, b2):
    src = edge_index[0]
    dst = edge_index[1]
    pad = E_PAD - E
    src_r = jnp.concatenate(
        [src, jnp.zeros((pad,), jnp.int32)]).reshape(NT, 2, CH2, B)
    dst_r = jnp.concatenate(
        [dst, jnp.full((pad,), N, jnp.int32)]).reshape(NT, 2, CH2, B)
    z128 = jnp.zeros((N_PAD, D_P), _f32)
    ones128 = jnp.ones((B, D_P), _f32)
    w2p = jnp.pad(W2, ((0, 0), (0, D2 - W2.shape[1])))
    b1r = b1.reshape(1, D_H)
    g1r = gamma1.reshape(1, D_H)
    be1r = beta1.reshape(1, D_H)
    b2r = jnp.pad(b2, (0, D2 - b2.shape[0])).reshape(1, D2)

    degp = _sc_hist(ones128, src_r, dst_r, z128)
    xw = pl.pallas_call(
        _tc_matmul,
        out_shape=jax.ShapeDtypeStruct((N, D_H), _f32),
    )(x, W1)

    y1, dinv = pl.pallas_call(
        _tc_scale,
        out_shape=[jax.ShapeDtypeStruct((N, D_H), _f32),
                   jax.ShapeDtypeStruct((N_PAD, D2), _f32)],
    )(degp, xw)

    y1p = jnp.pad(y1, ((0, 0), (0, D_P - D_H)))
    seg1p = _sc_scatter(y1p, src_r, dst_r, z128)

    z = pl.pallas_call(
        _tc_mid,
        out_shape=jax.ShapeDtypeStruct((N, D_H), _f32),
    )(seg1p, y1, dinv, b1r, g1r, be1r)

    zp = jnp.pad(z, ((0, 0), (0, D_P - D_H)))
    seg2p = _sc_scatter(zp, src_r, dst_r, z128)

    out8 = pl.pallas_call(
        _tc_final,
        out_shape=jax.ShapeDtypeStruct((N, D2), _f32),
    )(seg2p, z, dinv, w2p, b2r)

    return out8[:, :2]


# speedup vs baseline: 1.3732x; 1.3721x over previous
import functools

import jax
import jax.numpy as jnp
from jax import lax
from jax.experimental import pallas as pl
from jax.experimental.pallas import tpu as pltpu
from jax.experimental.pallas import tpu_sc as plsc

N = 10000
E = 320000
D_IN = 128
D_H = 64
D2 = 16
D_P = 128
NC = 2
NS = 16
NT = NC * NS
B = 128
CH = (E + NT * B - 1) // (NT * B)
E_PAD = NT * CH * B
N_PAD = 10112
RPT = N_PAD // NS

_f32 = jnp.float32


def _sc_scatter_add(D, gather):
    mesh = plsc.VectorSubcoreMesh(core_axis_name="c", subcore_axis_name="s")
    scratch = [
        pltpu.VMEM((CH, B), jnp.int32),
        pltpu.VMEM((CH, B), jnp.int32),
        pltpu.VMEM((B, D), _f32),
        pltpu.SemaphoreType.DMA,
        pltpu.VMEM_SHARED((N_PAD, D), _f32),
    ]

    @functools.partial(
        pl.kernel,
        out_type=jax.ShapeDtypeStruct((NC, N_PAD, D), _f32),
        mesh=mesh,
        scratch_types=scratch,
    )
    def k(vals_hbm, srcr_hbm, dstr_hbm, zeros_hbm, out_hbm,
          dst_v, src_v, rows_v, sem, accum):
        c = lax.axis_index("c")
        s = lax.axis_index("s")
        t = c * NS + s
        pltpu.sync_copy(dstr_hbm.at[t], dst_v)
        if gather:
            pltpu.sync_copy(srcr_hbm.at[t], src_v)
        else:
            pltpu.sync_copy(vals_hbm, rows_v)
        pltpu.sync_copy(zeros_hbm.at[pl.ds(s * RPT, RPT)],
                        accum.at[pl.ds(s * RPT, RPT)])
        plsc.subcore_barrier()

        def step(j, carry):
            if gather:
                pltpu.async_copy(vals_hbm.at[src_v.at[j]], rows_v, sem).wait()
            pltpu.sync_copy(rows_v, accum.at[dst_v.at[j]], add=True)
            return carry

        lax.fori_loop(0, CH, step, 0)
        plsc.subcore_barrier()
        pltpu.sync_copy(accum.at[pl.ds(s * RPT, RPT)],
                        out_hbm.at[c].at[pl.ds(s * RPT, RPT)])

    return k


_sc_hist = _sc_scatter_add(D_P, gather=False)
_sc_scatter = _sc_scatter_add(D_P, gather=True)

_zpad = None


def _tc_first(degp_ref, x_ref, w1_ref, y1p_ref, dinv_ref):
    deg = 1.0 + degp_ref[0][:, :D2] + degp_ref[1][:, :D2]
    dinv = lax.rsqrt(deg)
    dinv_ref[...] = dinv
    xw = jnp.dot(x_ref[...], w1_ref[...], preferred_element_type=_f32)
    y1 = xw * dinv[:N, 0:1]
    y1p_ref[...] = jnp.concatenate(
        [y1, jnp.zeros((N, D_P - D_H), _f32)], axis=1)


def _tc_mid(segp_ref, y1p_ref, dinv_ref, b1_ref, g1_ref, be1_ref, zp_ref):
    dcol = dinv_ref[...][:N, 0:1]
    y1 = y1p_ref[...][:, :D_H]
    seg = segp_ref[0][:N, :D_H] + segp_ref[1][:N, :D_H] + y1
    hpre = seg * dcol + b1_ref[...]
    mean = jnp.mean(hpre, axis=0, keepdims=True)
    var = jnp.mean((hpre - mean) ** 2, axis=0, keepdims=True)
    h = (hpre - mean) * lax.rsqrt(var + 1e-5) * g1_ref[...] + be1_ref[...]
    h = jnp.maximum(h, 0.0)
    z = h * dcol
    zp_ref[...] = jnp.concatenate(
        [z, jnp.zeros((N, D_P - D_H), _f32)], axis=1)


def _tc_final(segp_ref, zp_ref, dinv_ref, w2_ref, b2_ref, o_ref):
    dcol = dinv_ref[...][:N, 0:1]
    z = zp_ref[...][:, :D_H]
    seg = (segp_ref[0][:N, :D_H] + segp_ref[1][:N, :D_H] + z) * dcol
    o_ref[...] = jnp.dot(seg, w2_ref[...],
                         preferred_element_type=_f32) + b2_ref[...]


def kernel(x, edge_index, W1, b1, gamma1, beta1, W2, b2):
    src = edge_index[0]
    dst = edge_index[1]
    pad = E_PAD - E
    src_r = jnp.concatenate(
        [src, jnp.zeros((pad,), jnp.int32)]).reshape(NT, CH, B)
    dst_r = jnp.concatenate(
        [dst, jnp.full((pad,), N, jnp.int32)]).reshape(NT, CH, B)
    z128 = jnp.zeros((N_PAD, D_P), _f32)
    ones128 = jnp.ones((B, D_P), _f32)
    w2p = jnp.pad(W2, ((0, 0), (0, D2 - W2.shape[1])))
    b1r = b1.reshape(1, D_H)
    g1r = gamma1.reshape(1, D_H)
    be1r = beta1.reshape(1, D_H)
    b2r = jnp.pad(b2, (0, D2 - b2.shape[0])).reshape(1, D2)

    degp = _sc_hist(ones128, src_r, dst_r, z128)

    y1p, dinv = pl.pallas_call(
        _tc_first,
        out_shape=[jax.ShapeDtypeStruct((N, D_P), _f32),
                   jax.ShapeDtypeStruct((N_PAD, D2), _f32)],
    )(degp, x, W1)

    seg1p = _sc_scatter(y1p, src_r, dst_r, z128)

    zp = pl.pallas_call(
        _tc_mid,
        out_shape=jax.ShapeDtypeStruct((N, D_P), _f32),
    )(seg1p, y1p, dinv, b1r, g1r, be1r)

    seg2p = _sc_scatter(zp, src_r, dst_r, z128)

    out8 = pl.pallas_call(
        _tc_final,
        out_shape=jax.ShapeDtypeStruct((N, D2), _f32),
    )(seg2p, zp, dinv, w2p, b2r)

    return out8[:, :2]
